# baseline (speedup 1.0000x reference)
import functools

import jax
import jax.numpy as jnp
from jax import lax
from jax.experimental import pallas as pl
from jax.experimental.pallas import tpu as pltpu
from jax.experimental.pallas import tpu_sc as plsc

_B = 16384
_C = 1000
_K = 11468
_LMBDA = 0.5
_CB = 1024
_NG = _B // _CB


def _row_stats_kernel(w_ref, yh_ref, y_ref, o_ref):
    yv = y_ref[...]
    yh = yh_ref[...]
    m = jnp.max(yv, axis=0)
    row = lax.broadcasted_iota(jnp.int32, (_C, _CB), 0)
    t = jnp.min(jnp.where(yv == m[None, :], row, _C), axis=0)
    hit = row == t[None, :]
    v = jnp.sum(jnp.where(hit, yh, 0.0), axis=0)
    wrow = jnp.sum(jnp.where(hit, w_ref[...], 0.0), axis=0)
    d = jnp.sum(yh * yv, axis=0)
    o_ref[0, :] = -v + _LMBDA * (1.0 - d)
    o_ref[1, :] = -jnp.log(v)
    o_ref[2, :] = wrow


_NSUB = 16
_PT = _B // _NSUB
_NV = _PT // 16
_I32MIN = -2147483648


def _sc_select_kernel(s_hbm, l_hbm, w_hbm, out_hbm,
                      sbuf, lbuf, wbuf, kbuf, hist, comb, allb, sufb, binsb,
                      ebuf, obuf, shared_bins, shared_cnt, shared_part):
    cid = lax.axis_index("c")
    sid = lax.axis_index("s")

    @pl.when(cid == 0)
    def _body():
        base = sid * _PT
        pltpu.sync_copy(s_hbm.at[pl.ds(base, _PT)], sbuf)
        pltpu.sync_copy(l_hbm.at[pl.ds(base, _PT)], lbuf)
        pltpu.sync_copy(w_hbm.at[pl.ds(base, _PT)], wbuf)

        lane = lax.broadcasted_iota(jnp.int32, (16,), 0)
        izero = jnp.zeros((16,), jnp.int32)
        fzero = jnp.zeros((16,), jnp.float32)
        iones = jnp.ones((16,), jnp.int32)

        def mkkey(j, carry):
            v = sbuf[pl.ds(j * 16, 16)]
            b = lax.bitcast_convert_type(v, jnp.int32)
            key = jnp.where(
                b < 0,
                jnp.bitwise_xor(jnp.bitwise_not(b), jnp.int32(_I32MIN)), b)
            kbuf[pl.ds(j * 16, 16)] = key
            return carry
        lax.fori_loop(0, _NV, mkkey, 0)

        prefix = izero
        k_rem = jnp.full((16,), _K, jnp.int32)

        for r in range(4):
            shift = 24 - 8 * r
            hm = jnp.int32([0, -16777216, -65536, -256][r])

            def zh(i, carry):
                hist[pl.ds(i * 16, 16)] = izero
                return carry
            lax.fori_loop(0, 256, zh, 0)

            def scat(j, carry):
                key = kbuf[pl.ds(j * 16, 16)]
                act = (key & hm) == (prefix & hm)
                digit = lax.shift_right_logical(key, shift) & 255
                slot = lane * 256 + digit
                plsc.addupdate_scatter(hist, [slot], iones, mask=act)
                return carry
            lax.fori_loop(0, _NV, scat, 0)

            def lc(g, carry):
                acc = izero
                for j in range(16):
                    acc = acc + hist[pl.ds(j * 256 + g * 16, 16)]
                comb[pl.ds(g * 16, 16)] = acc
                return carry
            lax.fori_loop(0, 16, lc, 0)

            pltpu.sync_copy(comb, shared_bins.at[pl.ds(sid * 256, 256)])
            plsc.subcore_barrier()
            pltpu.sync_copy(shared_bins, allb)
            plsc.subcore_barrier()

            def gb(g, carry):
                acc = izero
                for t in range(16):
                    acc = acc + allb[pl.ds(t * 256 + g * 16, 16)]
                binsb[pl.ds(g * 16, 16)] = acc
                return carry
            lax.fori_loop(0, 16, gb, 0)

            carry = izero
            cnt_ge = izero
            for g in range(15, -1, -1):
                bv = binsb[pl.ds(g * 16, 16)]
                suf = lax.rev(plsc.cumsum(lax.rev(bv, (0,))), (0,)) + carry
                sufb[pl.ds(g * 16, 16)] = suf
                carry = carry + jnp.sum(bv)
                cnt_ge = cnt_ge + plsc.all_reduce_population_count(
                    suf >= k_rem)
            d_star = cnt_ge - 1
            a_d = plsc.load_gather(sufb, [d_star])
            b_d = plsc.load_gather(binsb, [d_star])
            k_rem = k_rem - (a_d - b_d)
            prefix = prefix | lax.shift_left(d_star, shift)

        tau = prefix
        need = k_rem

        def fsum(j, carry):
            numv, denv, eqc = carry
            key = kbuf[pl.ds(j * 16, 16)]
            lv = lbuf[pl.ds(j * 16, 16)]
            wv = wbuf[pl.ds(j * 16, 16)]
            gtm = key > tau
            numv = numv + jnp.where(gtm, wv * lv, 0.0)
            denv = denv + jnp.where(gtm, wv, 0.0)
            eqc = eqc + jnp.where(key == tau, 1, 0)
            return (numv, denv, eqc)
        numv, denv, eqcv = lax.fori_loop(0, _NV, fsum, (fzero, fzero, izero))
        eq_tot = jnp.sum(eqcv)

        ebuf[...] = jnp.broadcast_to(eq_tot, (16,))
        pltpu.sync_copy(ebuf, shared_cnt.at[pl.ds(sid * 16, 16)])
        plsc.subcore_barrier()
        pltpu.sync_copy(shared_cnt, allb.at[pl.ds(0, 256)])

        pre = izero
        for t in range(16):
            rowv = allb[pl.ds(t * 16, 16)]
            pre = pre + jnp.where(jnp.int32(t) < sid, rowv, izero)
        quota = jnp.minimum(jnp.maximum(need - pre, 0),
                            jnp.broadcast_to(eq_tot, (16,)))

        def feq(j, carry):
            numv, denv, cnt = carry
            key = kbuf[pl.ds(j * 16, 16)]
            lv = lbuf[pl.ds(j * 16, 16)]
            wv = wbuf[pl.ds(j * 16, 16)]
            eqm = key == tau
            eqi = jnp.where(eqm, 1, 0)
            rank = cnt + plsc.cumsum(eqi)
            inc = eqm & (rank <= quota)
            numv = numv + jnp.where(inc, wv * lv, 0.0)
            denv = denv + jnp.where(inc, wv, 0.0)
            cnt = cnt + jnp.sum(eqi)
            return (numv, denv, cnt)
        numv, denv, _ = lax.fori_loop(0, _NV, feq, (numv, denv, izero))

        num_s = jnp.sum(numv)
        den_s = jnp.sum(denv)
        obuf[...] = jnp.where(lane == 0, num_s,
                              jnp.where(lane == 1, den_s, 0.0))
        pltpu.sync_copy(obuf, shared_part.at[pl.ds(sid * 16, 16)])
        plsc.subcore_barrier()

        @pl.when(sid == 0)
        def _fin():
            pltpu.sync_copy(shared_part, lbuf.at[pl.ds(0, 256)])
            acc = fzero
            for t in range(16):
                acc = acc + lbuf[pl.ds(t * 16, 16)]
            wbuf[pl.ds(0, 16)] = acc
            n0 = plsc.load_gather(wbuf, [izero])
            d0 = plsc.load_gather(wbuf, [iones])
            obuf[...] = n0 / d0
            pltpu.sync_copy(obuf, out_hbm)


def _sc_select(s1, l1, w1):
    mesh = plsc.VectorSubcoreMesh(core_axis_name="c", subcore_axis_name="s")
    f = functools.partial(
        pl.kernel,
        out_type=jax.ShapeDtypeStruct((16,), jnp.float32),
        mesh=mesh,
        scratch_types=[
            pltpu.VMEM((_PT,), jnp.float32),
            pltpu.VMEM((_PT,), jnp.float32),
            pltpu.VMEM((_PT,), jnp.float32),
            pltpu.VMEM((_PT,), jnp.int32),
            pltpu.VMEM((4096,), jnp.int32),
            pltpu.VMEM((256,), jnp.int32),
            pltpu.VMEM((4096,), jnp.int32),
            pltpu.VMEM((256,), jnp.int32),
            pltpu.VMEM((256,), jnp.int32),
            pltpu.VMEM((16,), jnp.int32),
            pltpu.VMEM((16,), jnp.float32),
            pltpu.VMEM_SHARED((4096,), jnp.int32),
            pltpu.VMEM_SHARED((256,), jnp.int32),
            pltpu.VMEM_SHARED((256,), jnp.float32),
        ],
        compiler_params=pltpu.CompilerParams(needs_layout_passes=False),
    )(_sc_select_kernel)
    return f(s1, l1, w1)


def _select_kernel(s_ref, l_ref, w_ref, out_ref):
    s = s_ref[...]
    b = lax.bitcast_convert_type(s, jnp.int32)
    keys = jnp.where(b < 0,
                     jnp.bitwise_xor(jnp.bitwise_not(b), jnp.int32(-2147483648)),
                     b)

    def count_ge(thr):
        return jnp.sum((keys >= thr).astype(jnp.int32))

    def body(_, carry):
        lo, hi = carry
        d = hi - lo
        half = lax.shift_right_logical(d, 1) + jnp.bitwise_and(d, 1)
        mid = lo + half
        cond = count_ge(mid) >= _K
        return (jnp.where(cond, mid, lo), jnp.where(cond, hi, mid - 1))

    lo, _ = lax.fori_loop(0, 32, body,
                          (jnp.int32(-2147483648), jnp.int32(2147483647)))
    tau = lo
    gt = keys > tau
    eq = keys == tau
    n_gt = jnp.sum(gt.astype(jnp.int32))
    need = _K - n_gt

    lin = (lax.broadcasted_iota(jnp.int32, (128, 128), 0) * 128
           + lax.broadcasted_iota(jnp.int32, (128, 128), 1))

    def body2(_, carry):
        lo2, hi2 = carry
        mid = lax.shift_right_logical(lo2 + hi2, 1)
        cnt = jnp.sum((eq & (lin <= mid)).astype(jnp.int32))
        cond = cnt >= need
        return (jnp.where(cond, lo2, mid + 1), jnp.where(cond, mid, hi2))

    c, _ = lax.fori_loop(0, 14, body2, (jnp.int32(0), jnp.int32(_B - 1)))
    sel = gt | (eq & (lin <= c))

    w = w_ref[...]
    l = l_ref[...]
    num = jnp.sum(jnp.where(sel, w * l, 0.0))
    den = jnp.sum(jnp.where(sel, w, 0.0))
    out_ref[...] = jnp.broadcast_to(num / den, (1, 128))


def kernel(y_hat, y, weights):
    yh_t = y_hat.T
    y_t = y.T
    w2 = weights.reshape(_C, 1)
    slw = pl.pallas_call(
        _row_stats_kernel,
        grid=(_NG,),
        in_specs=[
            pl.BlockSpec((_C, 1), lambda g: (0, 0)),
            pl.BlockSpec((_C, _CB), lambda g: (0, g)),
            pl.BlockSpec((_C, _CB), lambda g: (0, g)),
        ],
        out_specs=pl.BlockSpec((3, _CB), lambda g: (0, g)),
        out_shape=jax.ShapeDtypeStruct((3, _B), jnp.float32),
    )(w2, yh_t, y_t)
    out = _sc_select(slw[0], slw[1], slw[2])
    return out[0]


# speedup vs baseline: 1.0513x; 1.0513x over previous
import functools

import jax
import jax.numpy as jnp
from jax import lax
from jax.experimental import pallas as pl
from jax.experimental.pallas import tpu as pltpu
from jax.experimental.pallas import tpu_sc as plsc

_B = 16384
_C = 1000
_K = 11468
_LMBDA = 0.5
_CB = 1024
_NG = _B // _CB


def _row_stats_kernel(w_ref, yh_ref, y_ref, o_ref):
    yv = y_ref[...]
    yh = yh_ref[...]
    m = jnp.max(yv, axis=0)
    row = lax.broadcasted_iota(jnp.int32, (_C, _CB), 0)
    t = jnp.min(jnp.where(yv == m[None, :], row, _C), axis=0)
    hit = row == t[None, :]
    v = jnp.sum(jnp.where(hit, yh, 0.0), axis=0)
    wrow = jnp.sum(jnp.where(hit, w_ref[...], 0.0), axis=0)
    d = jnp.sum(yh * yv, axis=0)
    o_ref[0, :] = -v + _LMBDA * (1.0 - d)
    o_ref[1, :] = -jnp.log(v)
    o_ref[2, :] = wrow


_NSUB = 16
_PT = _B // _NSUB
_NV = _PT // 16
_I32MIN = -2147483648


def _sc_select_kernel(slw_hbm, out_hbm,
                      sbuf, lbuf, wbuf, kbuf, hist, comb, allb, sufb, binsb,
                      ebuf, obuf, shared_bins, shared_cnt, shared_part):
    cid = lax.axis_index("c")
    sid = lax.axis_index("s")

    @pl.when(cid == 0)
    def _body():
        base = sid * _PT
        pltpu.sync_copy(slw_hbm.at[pl.ds(base, _PT)], sbuf)
        pltpu.sync_copy(slw_hbm.at[pl.ds(_B + base, _PT)], lbuf)
        pltpu.sync_copy(slw_hbm.at[pl.ds(2 * _B + base, _PT)], wbuf)

        lane = lax.broadcasted_iota(jnp.int32, (16,), 0)
        izero = jnp.zeros((16,), jnp.int32)
        fzero = jnp.zeros((16,), jnp.float32)
        iones = jnp.ones((16,), jnp.int32)

        def mkkey(j, carry):
            for u in range(4):
                v = sbuf[pl.ds((j * 4 + u) * 16, 16)]
                b = lax.bitcast_convert_type(v, jnp.int32)
                key = jnp.where(
                    b < 0,
                    jnp.bitwise_xor(jnp.bitwise_not(b), jnp.int32(_I32MIN)), b)
                kbuf[pl.ds((j * 4 + u) * 16, 16)] = key
            return carry
        lax.fori_loop(0, _NV // 4, mkkey, 0)

        prefix = izero
        k_rem = jnp.full((16,), _K, jnp.int32)

        def zh(i, carry):
            hist[pl.ds(i * 16, 16)] = izero
            return carry
        lax.fori_loop(0, 256, zh, 0)

        for r in range(4):
            shift = 24 - 8 * r
            hm = jnp.int32([0, -16777216, -65536, -256][r])

            def scat(j, carry):
                for u in range(4):
                    key = kbuf[pl.ds((j * 4 + u) * 16, 16)]
                    act = (key & hm) == (prefix & hm)
                    digit = lax.shift_right_logical(key, shift) & 255
                    slot = lane * 256 + digit
                    plsc.addupdate_scatter(hist, [slot], iones, mask=act)
                return carry
            lax.fori_loop(0, _NV // 4, scat, 0)

            def lc(g, carry):
                acc = izero
                for j in range(16):
                    acc = acc + hist[pl.ds(j * 256 + g * 16, 16)]
                    hist[pl.ds(j * 256 + g * 16, 16)] = izero
                comb[pl.ds(g * 16, 16)] = acc
                return carry
            lax.fori_loop(0, 16, lc, 0)

            pltpu.sync_copy(comb, shared_bins.at[pl.ds(sid * 256, 256)])
            plsc.subcore_barrier()
            pltpu.sync_copy(shared_bins, allb)
            plsc.subcore_barrier()

            def gb(g, carry):
                acc = izero
                for t in range(16):
                    acc = acc + allb[pl.ds(t * 256 + g * 16, 16)]
                binsb[pl.ds(g * 16, 16)] = acc
                return carry
            lax.fori_loop(0, 16, gb, 0)

            carry = izero
            cnt_ge = izero
            for g in range(15, -1, -1):
                bv = binsb[pl.ds(g * 16, 16)]
                suf = lax.rev(plsc.cumsum(lax.rev(bv, (0,))), (0,)) + carry
                sufb[pl.ds(g * 16, 16)] = suf
                carry = carry + jnp.sum(bv)
                cnt_ge = cnt_ge + plsc.all_reduce_population_count(
                    suf >= k_rem)
            d_star = cnt_ge - 1
            a_d = plsc.load_gather(sufb, [d_star])
            b_d = plsc.load_gather(binsb, [d_star])
            k_rem = k_rem - (a_d - b_d)
            prefix = prefix | lax.shift_left(d_star, shift)

        tau = prefix
        need = k_rem

        def fsum(j, carry):
            numv, denv, eqc = carry
            for u in range(4):
                o = (j * 4 + u) * 16
                key = kbuf[pl.ds(o, 16)]
                lv = lbuf[pl.ds(o, 16)]
                wv = wbuf[pl.ds(o, 16)]
                gtm = key > tau
                numv = numv + jnp.where(gtm, wv * lv, 0.0)
                denv = denv + jnp.where(gtm, wv, 0.0)
                eqc = eqc + jnp.where(key == tau, 1, 0)
            return (numv, denv, eqc)
        numv, denv, eqcv = lax.fori_loop(0, _NV // 4, fsum,
                                         (fzero, fzero, izero))
        eq_tot = jnp.sum(eqcv)

        ebuf[...] = jnp.broadcast_to(eq_tot, (16,))
        pltpu.sync_copy(ebuf, shared_cnt.at[pl.ds(sid * 16, 16)])
        plsc.subcore_barrier()
        pltpu.sync_copy(shared_cnt, allb.at[pl.ds(0, 256)])

        pre = izero
        for t in range(16):
            rowv = allb[pl.ds(t * 16, 16)]
            pre = pre + jnp.where(jnp.int32(t) < sid, rowv, izero)
        quota = jnp.minimum(jnp.maximum(need - pre, 0),
                            jnp.broadcast_to(eq_tot, (16,)))

        def feq(j, carry):
            numv, denv, cnt = carry
            key = kbuf[pl.ds(j * 16, 16)]
            lv = lbuf[pl.ds(j * 16, 16)]
            wv = wbuf[pl.ds(j * 16, 16)]
            eqm = key == tau
            eqi = jnp.where(eqm, 1, 0)
            rank = cnt + plsc.cumsum(eqi)
            inc = eqm & (rank <= quota)
            numv = numv + jnp.where(inc, wv * lv, 0.0)
            denv = denv + jnp.where(inc, wv, 0.0)
            cnt = cnt + jnp.sum(eqi)
            return (numv, denv, cnt)
        numv, denv, _ = lax.fori_loop(0, _NV, feq, (numv, denv, izero))

        num_s = jnp.sum(numv)
        den_s = jnp.sum(denv)
        obuf[...] = jnp.where(lane == 0, num_s,
                              jnp.where(lane == 1, den_s, 0.0))
        pltpu.sync_copy(obuf, shared_part.at[pl.ds(sid * 16, 16)])
        plsc.subcore_barrier()

        @pl.when(sid == 0)
        def _fin():
            pltpu.sync_copy(shared_part, lbuf.at[pl.ds(0, 256)])
            acc = fzero
            for t in range(16):
                acc = acc + lbuf[pl.ds(t * 16, 16)]
            wbuf[pl.ds(0, 16)] = acc
            n0 = plsc.load_gather(wbuf, [izero])
            d0 = plsc.load_gather(wbuf, [iones])
            obuf[...] = n0 / d0
            pltpu.sync_copy(obuf, out_hbm)


def _sc_select(slw):
    mesh = plsc.VectorSubcoreMesh(core_axis_name="c", subcore_axis_name="s")
    f = functools.partial(
        pl.kernel,
        out_type=jax.ShapeDtypeStruct((16,), jnp.float32),
        mesh=mesh,
        scratch_types=[
            pltpu.VMEM((_PT,), jnp.float32),
            pltpu.VMEM((_PT,), jnp.float32),
            pltpu.VMEM((_PT,), jnp.float32),
            pltpu.VMEM((_PT,), jnp.int32),
            pltpu.VMEM((4096,), jnp.int32),
            pltpu.VMEM((256,), jnp.int32),
            pltpu.VMEM((4096,), jnp.int32),
            pltpu.VMEM((256,), jnp.int32),
            pltpu.VMEM((256,), jnp.int32),
            pltpu.VMEM((16,), jnp.int32),
            pltpu.VMEM((16,), jnp.float32),
            pltpu.VMEM_SHARED((4096,), jnp.int32),
            pltpu.VMEM_SHARED((256,), jnp.int32),
            pltpu.VMEM_SHARED((256,), jnp.float32),
        ],
        compiler_params=pltpu.CompilerParams(needs_layout_passes=False),
    )(_sc_select_kernel)
    return f(slw.reshape(3 * _B))


def _select_kernel(s_ref, l_ref, w_ref, out_ref):
    s = s_ref[...]
    b = lax.bitcast_convert_type(s, jnp.int32)
    keys = jnp.where(b < 0,
                     jnp.bitwise_xor(jnp.bitwise_not(b), jnp.int32(-2147483648)),
                     b)

    def count_ge(thr):
        return jnp.sum((keys >= thr).astype(jnp.int32))

    def body(_, carry):
        lo, hi = carry
        d = hi - lo
        half = lax.shift_right_logical(d, 1) + jnp.bitwise_and(d, 1)
        mid = lo + half
        cond = count_ge(mid) >= _K
        return (jnp.where(cond, mid, lo), jnp.where(cond, hi, mid - 1))

    lo, _ = lax.fori_loop(0, 32, body,
                          (jnp.int32(-2147483648), jnp.int32(2147483647)))
    tau = lo
    gt = keys > tau
    eq = keys == tau
    n_gt = jnp.sum(gt.astype(jnp.int32))
    need = _K - n_gt

    lin = (lax.broadcasted_iota(jnp.int32, (128, 128), 0) * 128
           + lax.broadcasted_iota(jnp.int32, (128, 128), 1))

    def body2(_, carry):
        lo2, hi2 = carry
        mid = lax.shift_right_logical(lo2 + hi2, 1)
        cnt = jnp.sum((eq & (lin <= mid)).astype(jnp.int32))
        cond = cnt >= need
        return (jnp.where(cond, lo2, mid + 1), jnp.where(cond, mid, hi2))

    c, _ = lax.fori_loop(0, 14, body2, (jnp.int32(0), jnp.int32(_B - 1)))
    sel = gt | (eq & (lin <= c))

    w = w_ref[...]
    l = l_ref[...]
    num = jnp.sum(jnp.where(sel, w * l, 0.0))
    den = jnp.sum(jnp.where(sel, w, 0.0))
    out_ref[...] = jnp.broadcast_to(num / den, (1, 128))


def kernel(y_hat, y, weights):
    yh_t = y_hat.T
    y_t = y.T
    w2 = weights.reshape(_C, 1)
    slw = pl.pallas_call(
        _row_stats_kernel,
        grid=(_NG,),
        in_specs=[
            pl.BlockSpec((_C, 1), lambda g: (0, 0)),
            pl.BlockSpec((_C, _CB), lambda g: (0, g)),
            pl.BlockSpec((_C, _CB), lambda g: (0, g)),
        ],
        out_specs=pl.BlockSpec((3, _CB), lambda g: (0, g)),
        out_shape=jax.ShapeDtypeStruct((3, _B), jnp.float32),
    )(w2, yh_t, y_t)
    out = _sc_select(slw)
    return out[0]


# speedup vs baseline: 1.0832x; 1.0304x over previous
import functools

import jax
import jax.numpy as jnp
from jax import lax
from jax.experimental import pallas as pl
from jax.experimental.pallas import tpu as pltpu
from jax.experimental.pallas import tpu_sc as plsc

_B = 16384
_C = 1000
_K = 11468
_LMBDA = 0.5
_CB = 1024
_NG = _B // _CB


def _row_stats_kernel(w_ref, yh_ref, y_ref, o_ref):
    yv = y_ref[...]
    yh = yh_ref[...]
    m = jnp.max(yv, axis=0)
    row = lax.broadcasted_iota(jnp.int32, (_C, _CB), 0)
    t = jnp.min(jnp.where(yv == m[None, :], row, _C), axis=0)
    hit = row == t[None, :]
    v = jnp.sum(jnp.where(hit, yh, 0.0), axis=0)
    wrow = jnp.sum(jnp.where(hit, w_ref[...], 0.0), axis=0)
    d = jnp.sum(yh * yv, axis=0)
    o_ref[0, :] = -v + _LMBDA * (1.0 - d)
    o_ref[1, :] = -jnp.log(v)
    o_ref[2, :] = wrow


_NSUB = 16
_PT = _B // _NSUB
_NV = _PT // 16
_I32MIN = -2147483648


def _sc_select_kernel(slw_hbm, out_hbm,
                      sbuf, lbuf, wbuf, kbuf, comb, allb, sufb, binsb,
                      ebuf, obuf, shared_bins, shared_cnt, shared_part):
    cid = lax.axis_index("c")
    sid = lax.axis_index("s")

    @pl.when(cid == 0)
    def _body():
        base = sid * _PT
        pltpu.sync_copy(slw_hbm.at[pl.ds(base, _PT)], sbuf)
        pltpu.sync_copy(slw_hbm.at[pl.ds(_B + base, _PT)], lbuf)
        pltpu.sync_copy(slw_hbm.at[pl.ds(2 * _B + base, _PT)], wbuf)

        lane = lax.broadcasted_iota(jnp.int32, (16,), 0)
        izero = jnp.zeros((16,), jnp.int32)
        fzero = jnp.zeros((16,), jnp.float32)
        iones = jnp.ones((16,), jnp.int32)

        def mkkey(j, carry):
            for u in range(4):
                v = sbuf[pl.ds((j * 4 + u) * 16, 16)]
                b = lax.bitcast_convert_type(v, jnp.int32)
                key = jnp.where(
                    b < 0,
                    jnp.bitwise_xor(jnp.bitwise_not(b), jnp.int32(_I32MIN)), b)
                kbuf[pl.ds((j * 4 + u) * 16, 16)] = key
            return carry
        lax.fori_loop(0, _NV // 4, mkkey, 0)

        prefix = izero
        k_rem = jnp.full((16,), _K, jnp.int32)

        def zh(i, carry):
            comb[pl.ds(i * 16, 16)] = izero
            return carry
        lax.fori_loop(0, 16, zh, 0)

        for r in range(4):
            shift = 24 - 8 * r
            hm = jnp.int32([0, -16777216, -65536, -256][r])
            half = (r % 2) * 4096

            def scat(j, carry):
                for u in range(4):
                    key = kbuf[pl.ds((j * 4 + u) * 16, 16)]
                    act = (key & hm) == (prefix & hm)
                    digit = lax.shift_right_logical(key, shift) & 255
                    plsc.addupdate_scatter(comb, [digit], iones, mask=act)
                return carry
            lax.fori_loop(0, _NV // 4, scat, 0)

            pltpu.sync_copy(comb, shared_bins.at[pl.ds(half + sid * 256, 256)])
            lax.fori_loop(0, 16, zh, 0)
            plsc.subcore_barrier()
            pltpu.sync_copy(shared_bins.at[pl.ds(half, 4096)], allb)

            def gb(g, carry):
                acc = izero
                for t in range(16):
                    acc = acc + allb[pl.ds(t * 256 + g * 16, 16)]
                binsb[pl.ds(g * 16, 16)] = acc
                return carry
            lax.fori_loop(0, 16, gb, 0)

            carry = izero
            cnt_ge = izero
            for g in range(15, -1, -1):
                bv = binsb[pl.ds(g * 16, 16)]
                suf = lax.rev(plsc.cumsum(lax.rev(bv, (0,))), (0,)) + carry
                sufb[pl.ds(g * 16, 16)] = suf
                carry = carry + jnp.sum(bv)
                cnt_ge = cnt_ge + plsc.all_reduce_population_count(
                    suf >= k_rem)
            d_star = cnt_ge - 1
            a_d = plsc.load_gather(sufb, [d_star])
            b_d = plsc.load_gather(binsb, [d_star])
            k_rem = k_rem - (a_d - b_d)
            prefix = prefix | lax.shift_left(d_star, shift)

        tau = prefix
        need = k_rem

        def fsum(j, carry):
            numv, denv, eqc = carry
            for u in range(4):
                o = (j * 4 + u) * 16
                key = kbuf[pl.ds(o, 16)]
                lv = lbuf[pl.ds(o, 16)]
                wv = wbuf[pl.ds(o, 16)]
                gtm = key > tau
                numv = numv + jnp.where(gtm, wv * lv, 0.0)
                denv = denv + jnp.where(gtm, wv, 0.0)
                eqc = eqc + jnp.where(key == tau, 1, 0)
            return (numv, denv, eqc)
        numv, denv, eqcv = lax.fori_loop(0, _NV // 4, fsum,
                                         (fzero, fzero, izero))
        eq_tot = jnp.sum(eqcv)

        ebuf[...] = jnp.broadcast_to(eq_tot, (16,))
        pltpu.sync_copy(ebuf, shared_cnt.at[pl.ds(sid * 16, 16)])
        plsc.subcore_barrier()
        pltpu.sync_copy(shared_cnt, allb.at[pl.ds(0, 256)])

        pre = izero
        for t in range(16):
            rowv = allb[pl.ds(t * 16, 16)]
            pre = pre + jnp.where(jnp.int32(t) < sid, rowv, izero)
        quota = jnp.minimum(jnp.maximum(need - pre, 0),
                            jnp.broadcast_to(eq_tot, (16,)))

        def feq(j, carry):
            numv, denv, cnt = carry
            key = kbuf[pl.ds(j * 16, 16)]
            lv = lbuf[pl.ds(j * 16, 16)]
            wv = wbuf[pl.ds(j * 16, 16)]
            eqm = key == tau
            eqi = jnp.where(eqm, 1, 0)
            rank = cnt + plsc.cumsum(eqi)
            inc = eqm & (rank <= quota)
            numv = numv + jnp.where(inc, wv * lv, 0.0)
            denv = denv + jnp.where(inc, wv, 0.0)
            cnt = cnt + jnp.sum(eqi)
            return (numv, denv, cnt)
        numv, denv, _ = lax.fori_loop(0, _NV, feq, (numv, denv, izero))

        num_s = jnp.sum(numv)
        den_s = jnp.sum(denv)
        obuf[...] = jnp.where(lane == 0, num_s,
                              jnp.where(lane == 1, den_s, 0.0))
        pltpu.sync_copy(obuf, shared_part.at[pl.ds(sid * 16, 16)])
        plsc.subcore_barrier()

        @pl.when(sid == 0)
        def _fin():
            pltpu.sync_copy(shared_part, lbuf.at[pl.ds(0, 256)])
            acc = fzero
            for t in range(16):
                acc = acc + lbuf[pl.ds(t * 16, 16)]
            wbuf[pl.ds(0, 16)] = acc
            n0 = plsc.load_gather(wbuf, [izero])
            d0 = plsc.load_gather(wbuf, [iones])
            obuf[...] = n0 / d0
            pltpu.sync_copy(obuf, out_hbm)


def _sc_select(slw):
    mesh = plsc.VectorSubcoreMesh(core_axis_name="c", subcore_axis_name="s")
    f = functools.partial(
        pl.kernel,
        out_type=jax.ShapeDtypeStruct((16,), jnp.float32),
        mesh=mesh,
        scratch_types=[
            pltpu.VMEM((_PT,), jnp.float32),
            pltpu.VMEM((_PT,), jnp.float32),
            pltpu.VMEM((_PT,), jnp.float32),
            pltpu.VMEM((_PT,), jnp.int32),
            pltpu.VMEM((256,), jnp.int32),
            pltpu.VMEM((4096,), jnp.int32),
            pltpu.VMEM((256,), jnp.int32),
            pltpu.VMEM((256,), jnp.int32),
            pltpu.VMEM((16,), jnp.int32),
            pltpu.VMEM((16,), jnp.float32),
            pltpu.VMEM_SHARED((8192,), jnp.int32),
            pltpu.VMEM_SHARED((256,), jnp.int32),
            pltpu.VMEM_SHARED((256,), jnp.float32),
        ],
        compiler_params=pltpu.CompilerParams(needs_layout_passes=False),
    )(_sc_select_kernel)
    return f(slw.reshape(3 * _B))


def _select_kernel(s_ref, l_ref, w_ref, out_ref):
    s = s_ref[...]
    b = lax.bitcast_convert_type(s, jnp.int32)
    keys = jnp.where(b < 0,
                     jnp.bitwise_xor(jnp.bitwise_not(b), jnp.int32(-2147483648)),
                     b)

    def count_ge(thr):
        return jnp.sum((keys >= thr).astype(jnp.int32))

    def body(_, carry):
        lo, hi = carry
        d = hi - lo
        half = lax.shift_right_logical(d, 1) + jnp.bitwise_and(d, 1)
        mid = lo + half
        cond = count_ge(mid) >= _K
        return (jnp.where(cond, mid, lo), jnp.where(cond, hi, mid - 1))

    lo, _ = lax.fori_loop(0, 32, body,
                          (jnp.int32(-2147483648), jnp.int32(2147483647)))
    tau = lo
    gt = keys > tau
    eq = keys == tau
    n_gt = jnp.sum(gt.astype(jnp.int32))
    need = _K - n_gt

    lin = (lax.broadcasted_iota(jnp.int32, (128, 128), 0) * 128
           + lax.broadcasted_iota(jnp.int32, (128, 128), 1))

    def body2(_, carry):
        lo2, hi2 = carry
        mid = lax.shift_right_logical(lo2 + hi2, 1)
        cnt = jnp.sum((eq & (lin <= mid)).astype(jnp.int32))
        cond = cnt >= need
        return (jnp.where(cond, lo2, mid + 1), jnp.where(cond, mid, hi2))

    c, _ = lax.fori_loop(0, 14, body2, (jnp.int32(0), jnp.int32(_B - 1)))
    sel = gt | (eq & (lin <= c))

    w = w_ref[...]
    l = l_ref[...]
    num = jnp.sum(jnp.where(sel, w * l, 0.0))
    den = jnp.sum(jnp.where(sel, w, 0.0))
    out_ref[...] = jnp.broadcast_to(num / den, (1, 128))


def kernel(y_hat, y, weights):
    yh_t = y_hat.T
    y_t = y.T
    w2 = weights.reshape(_C, 1)
    slw = pl.pallas_call(
        _row_stats_kernel,
        grid=(_NG,),
        in_specs=[
            pl.BlockSpec((_C, 1), lambda g: (0, 0)),
            pl.BlockSpec((_C, _CB), lambda g: (0, g)),
            pl.BlockSpec((_C, _CB), lambda g: (0, g)),
        ],
        out_specs=pl.BlockSpec((3, _CB), lambda g: (0, g)),
        out_shape=jax.ShapeDtypeStruct((3, _B), jnp.float32),
    )(w2, yh_t, y_t)
    out = _sc_select(slw)
    return out[0]


# speedup vs baseline: 1.1088x; 1.0236x over previous
import functools

import jax
import jax.numpy as jnp
from jax import lax
from jax.experimental import pallas as pl
from jax.experimental.pallas import tpu as pltpu
from jax.experimental.pallas import tpu_sc as plsc

_B = 16384
_C = 1000
_K = 11468
_LMBDA = 0.5
_CB = 1024
_NG = _B // _CB


def _row_stats_kernel(w_ref, yh_ref, y_ref, s_ref, l_ref, wr_ref):
    yv = y_ref[...]
    yh = yh_ref[...]
    m = jnp.max(yv, axis=0)
    row = lax.broadcasted_iota(jnp.int32, (_C, _CB), 0)
    t = jnp.min(jnp.where(yv == m[None, :], row, _C), axis=0)
    hit = row == t[None, :]
    v = jnp.sum(jnp.where(hit, yh, 0.0), axis=0)
    wrow = jnp.sum(jnp.where(hit, w_ref[...], 0.0), axis=0)
    d = jnp.sum(yh * yv, axis=0)
    s_ref[...] = -v + _LMBDA * (1.0 - d)
    l_ref[...] = -jnp.log(v)
    wr_ref[...] = wrow


_NSUB = 16
_PT = _B // _NSUB
_NV = _PT // 16
_I32MIN = -2147483648


def _sc_select_kernel(s_hbm, l_hbm, w_hbm, out_hbm,
                      sbuf, lbuf, wbuf, kbuf, comb, allb, sufb, binsb,
                      ebuf, obuf, shared_bins, shared_cnt, shared_part):
    cid = lax.axis_index("c")
    sid = lax.axis_index("s")

    @pl.when(cid == 0)
    def _body():
        base = sid * _PT
        pltpu.sync_copy(s_hbm.at[pl.ds(base, _PT)], sbuf)
        pltpu.sync_copy(l_hbm.at[pl.ds(base, _PT)], lbuf)
        pltpu.sync_copy(w_hbm.at[pl.ds(base, _PT)], wbuf)

        lane = lax.broadcasted_iota(jnp.int32, (16,), 0)
        izero = jnp.zeros((16,), jnp.int32)
        fzero = jnp.zeros((16,), jnp.float32)
        iones = jnp.ones((16,), jnp.int32)

        def mkkey(j, carry):
            for u in range(4):
                v = sbuf[pl.ds((j * 4 + u) * 16, 16)]
                b = lax.bitcast_convert_type(v, jnp.int32)
                key = jnp.where(
                    b < 0,
                    jnp.bitwise_xor(jnp.bitwise_not(b), jnp.int32(_I32MIN)), b)
                kbuf[pl.ds((j * 4 + u) * 16, 16)] = key
            return carry
        lax.fori_loop(0, _NV // 4, mkkey, 0)

        prefix = izero
        k_rem = jnp.full((16,), _K, jnp.int32)

        def zh(i, carry):
            comb[pl.ds(i * 16, 16)] = izero
            return carry
        lax.fori_loop(0, 16, zh, 0)

        for r in range(4):
            shift = 24 - 8 * r
            hm = jnp.int32([0, -16777216, -65536, -256][r])
            half = (r % 2) * 4096

            def scat(j, carry):
                for u in range(4):
                    key = kbuf[pl.ds((j * 4 + u) * 16, 16)]
                    act = (key & hm) == (prefix & hm)
                    digit = lax.shift_right_logical(key, shift) & 255
                    plsc.addupdate_scatter(comb, [digit], iones, mask=act)
                return carry
            lax.fori_loop(0, _NV // 4, scat, 0)

            pltpu.sync_copy(comb, shared_bins.at[pl.ds(half + sid * 256, 256)])
            lax.fori_loop(0, 16, zh, 0)
            plsc.subcore_barrier()
            pltpu.sync_copy(shared_bins.at[pl.ds(half, 4096)], allb)

            def gb(g, carry):
                acc = izero
                for t in range(16):
                    acc = acc + allb[pl.ds(t * 256 + g * 16, 16)]
                binsb[pl.ds(g * 16, 16)] = acc
                return carry
            lax.fori_loop(0, 16, gb, 0)

            carry = izero
            cnt_ge = izero
            for g in range(15, -1, -1):
                bv = binsb[pl.ds(g * 16, 16)]
                suf = lax.rev(plsc.cumsum(lax.rev(bv, (0,))), (0,)) + carry
                sufb[pl.ds(g * 16, 16)] = suf
                carry = carry + jnp.sum(bv)
                cnt_ge = cnt_ge + plsc.all_reduce_population_count(
                    suf >= k_rem)
            d_star = cnt_ge - 1
            a_d = plsc.load_gather(sufb, [d_star])
            b_d = plsc.load_gather(binsb, [d_star])
            k_rem = k_rem - (a_d - b_d)
            prefix = prefix | lax.shift_left(d_star, shift)

        tau = prefix
        need = k_rem

        def fsum(j, carry):
            numv, denv, eqc = carry
            for u in range(4):
                o = (j * 4 + u) * 16
                key = kbuf[pl.ds(o, 16)]
                lv = lbuf[pl.ds(o, 16)]
                wv = wbuf[pl.ds(o, 16)]
                gtm = key > tau
                numv = numv + jnp.where(gtm, wv * lv, 0.0)
                denv = denv + jnp.where(gtm, wv, 0.0)
                eqc = eqc + jnp.where(key == tau, 1, 0)
            return (numv, denv, eqc)
        numv, denv, eqcv = lax.fori_loop(0, _NV // 4, fsum,
                                         (fzero, fzero, izero))
        eq_tot = jnp.sum(eqcv)

        ebuf[...] = jnp.broadcast_to(eq_tot, (16,))
        pltpu.sync_copy(ebuf, shared_cnt.at[pl.ds(sid * 16, 16)])
        plsc.subcore_barrier()
        pltpu.sync_copy(shared_cnt, allb.at[pl.ds(0, 256)])

        pre = izero
        for t in range(16):
            rowv = allb[pl.ds(t * 16, 16)]
            pre = pre + jnp.where(jnp.int32(t) < sid, rowv, izero)
        quota = jnp.minimum(jnp.maximum(need - pre, 0),
                            jnp.broadcast_to(eq_tot, (16,)))

        def feq(j, carry):
            numv, denv, cnt = carry
            key = kbuf[pl.ds(j * 16, 16)]
            lv = lbuf[pl.ds(j * 16, 16)]
            wv = wbuf[pl.ds(j * 16, 16)]
            eqm = key == tau
            eqi = jnp.where(eqm, 1, 0)
            rank = cnt + plsc.cumsum(eqi)
            inc = eqm & (rank <= quota)
            numv = numv + jnp.where(inc, wv * lv, 0.0)
            denv = denv + jnp.where(inc, wv, 0.0)
            cnt = cnt + jnp.sum(eqi)
            return (numv, denv, cnt)
        numv, denv, _ = lax.fori_loop(0, _NV, feq, (numv, denv, izero))

        num_s = jnp.sum(numv)
        den_s = jnp.sum(denv)
        obuf[...] = jnp.where(lane == 0, num_s,
                              jnp.where(lane == 1, den_s, 0.0))
        pltpu.sync_copy(obuf, shared_part.at[pl.ds(sid * 16, 16)])
        plsc.subcore_barrier()

        @pl.when(sid == 0)
        def _fin():
            pltpu.sync_copy(shared_part, lbuf.at[pl.ds(0, 256)])
            acc = fzero
            for t in range(16):
                acc = acc + lbuf[pl.ds(t * 16, 16)]
            wbuf[pl.ds(0, 16)] = acc
            n0 = plsc.load_gather(wbuf, [izero])
            d0 = plsc.load_gather(wbuf, [iones])
            obuf[...] = n0 / d0
            pltpu.sync_copy(obuf, out_hbm)


def _sc_select(s1, l1, w1):
    mesh = plsc.VectorSubcoreMesh(core_axis_name="c", subcore_axis_name="s")
    f = functools.partial(
        pl.kernel,
        out_type=jax.ShapeDtypeStruct((16,), jnp.float32),
        mesh=mesh,
        scratch_types=[
            pltpu.VMEM((_PT,), jnp.float32),
            pltpu.VMEM((_PT,), jnp.float32),
            pltpu.VMEM((_PT,), jnp.float32),
            pltpu.VMEM((_PT,), jnp.int32),
            pltpu.VMEM((256,), jnp.int32),
            pltpu.VMEM((4096,), jnp.int32),
            pltpu.VMEM((256,), jnp.int32),
            pltpu.VMEM((256,), jnp.int32),
            pltpu.VMEM((16,), jnp.int32),
            pltpu.VMEM((16,), jnp.float32),
            pltpu.VMEM_SHARED((8192,), jnp.int32),
            pltpu.VMEM_SHARED((256,), jnp.int32),
            pltpu.VMEM_SHARED((256,), jnp.float32),
        ],
        compiler_params=pltpu.CompilerParams(needs_layout_passes=False),
    )(_sc_select_kernel)
    return f(s1, l1, w1)


def _select_kernel(s_ref, l_ref, w_ref, out_ref):
    s = s_ref[...]
    b = lax.bitcast_convert_type(s, jnp.int32)
    keys = jnp.where(b < 0,
                     jnp.bitwise_xor(jnp.bitwise_not(b), jnp.int32(-2147483648)),
                     b)

    def count_ge(thr):
        return jnp.sum((keys >= thr).astype(jnp.int32))

    def body(_, carry):
        lo, hi = carry
        d = hi - lo
        half = lax.shift_right_logical(d, 1) + jnp.bitwise_and(d, 1)
        mid = lo + half
        cond = count_ge(mid) >= _K
        return (jnp.where(cond, mid, lo), jnp.where(cond, hi, mid - 1))

    lo, _ = lax.fori_loop(0, 32, body,
                          (jnp.int32(-2147483648), jnp.int32(2147483647)))
    tau = lo
    gt = keys > tau
    eq = keys == tau
    n_gt = jnp.sum(gt.astype(jnp.int32))
    need = _K - n_gt

    lin = (lax.broadcasted_iota(jnp.int32, (128, 128), 0) * 128
           + lax.broadcasted_iota(jnp.int32, (128, 128), 1))

    def body2(_, carry):
        lo2, hi2 = carry
        mid = lax.shift_right_logical(lo2 + hi2, 1)
        cnt = jnp.sum((eq & (lin <= mid)).astype(jnp.int32))
        cond = cnt >= need
        return (jnp.where(cond, lo2, mid + 1), jnp.where(cond, mid, hi2))

    c, _ = lax.fori_loop(0, 14, body2, (jnp.int32(0), jnp.int32(_B - 1)))
    sel = gt | (eq & (lin <= c))

    w = w_ref[...]
    l = l_ref[...]
    num = jnp.sum(jnp.where(sel, w * l, 0.0))
    den = jnp.sum(jnp.where(sel, w, 0.0))
    out_ref[...] = jnp.broadcast_to(num / den, (1, 128))


def kernel(y_hat, y, weights):
    yh_t = y_hat.T
    y_t = y.T
    w2 = weights.reshape(_C, 1)
    slw = pl.pallas_call(
        _row_stats_kernel,
        grid=(_NG,),
        in_specs=[
            pl.BlockSpec((_C, 1), lambda g: (0, 0)),
            pl.BlockSpec((_C, _CB), lambda g: (0, g)),
            pl.BlockSpec((_C, _CB), lambda g: (0, g)),
        ],
        out_specs=[
            pl.BlockSpec((_CB,), lambda g: (g,)),
            pl.BlockSpec((_CB,), lambda g: (g,)),
            pl.BlockSpec((_CB,), lambda g: (g,)),
        ],
        out_shape=[
            jax.ShapeDtypeStruct((_B,), jnp.float32),
            jax.ShapeDtypeStruct((_B,), jnp.float32),
            jax.ShapeDtypeStruct((_B,), jnp.float32),
        ],
    )(w2, yh_t, y_t)
    out = _sc_select(*slw)
    return out[0]
